# baseline (speedup 1.0000x reference)
import functools
import math

import jax
import jax.numpy as jnp
from jax import lax
from jax.experimental import pallas as pl
from jax.experimental.pallas import tpu as pltpu
from jax.experimental.pallas import tpu_sc as plsc

B = 4096
L = 200
D = 64
N = B * L
NC = 2
NS = 16
NW = NC * NS
N_PER_W = N // NW
K = 512
NCHUNK = N_PER_W // K
SCALE = math.sqrt(D)


def _sc_body(tokens_hbm, table_hbm, out_hbm, idx_v, buf, sem):
    wid = lax.axis_index("s") * NC + lax.axis_index("c")
    base = wid * N_PER_W
    pltpu.sync_copy(tokens_hbm.at[pl.ds(base, N_PER_W)], idx_v)

    def chunk(g, carry):
        pltpu.async_copy(
            table_hbm.at[idx_v.at[pl.ds(g * K, K)]], buf, sem
        ).wait()

        def scale_row(i, c2):
            for j in range(D // 16):
                sl = pl.ds(j * 16, 16)
                buf[i, sl] = buf[i, sl] * SCALE
            return c2

        lax.fori_loop(0, K, scale_row, 0)
        pltpu.sync_copy(buf, out_hbm.at[pl.ds(base + g * K, K)])
        return carry

    lax.fori_loop(0, NCHUNK, chunk, 0)


@jax.jit
def kernel(tokens, table):
    idx = tokens.reshape(-1).astype(jnp.int32)
    call = pl.kernel(
        _sc_body,
        mesh=plsc.VectorSubcoreMesh(core_axis_name="c", subcore_axis_name="s"),
        out_type=jax.ShapeDtypeStruct((N, D), jnp.float32),
        scratch_types=[
            pltpu.VMEM((N_PER_W,), jnp.int32),
            pltpu.VMEM((K, D), jnp.float32),
            pltpu.SemaphoreType.DMA,
        ],
        compiler_params=pltpu.CompilerParams(use_tc_tiling_on_sc=False),
    )
    out = call(idx, table)
    return out.reshape(B, L, D)


# speedup vs baseline: 1.1079x; 1.1079x over previous
import math

import jax
import jax.numpy as jnp
from jax import lax
from jax.experimental import pallas as pl
from jax.experimental.pallas import tpu as pltpu
from jax.experimental.pallas import tpu_sc as plsc

B = 4096
L = 200
D = 64
N = B * L
NC = 2
NS = 16
NW = NC * NS
N_PER_W = N // NW
K = 256
NCHUNK = N_PER_W // K
NPAIR = NCHUNK // 2
SCALE = math.sqrt(D)
ROWS_PER_STEP = 8


def _sc_body(tokens_hbm, table_hbm, out_hbm,
             idx_v, g0, g1, w0, w1, gs0, gs1, ws0, ws1):
    wid = lax.axis_index("s") * NC + lax.axis_index("c")
    base = wid * N_PER_W
    pltpu.sync_copy(tokens_hbm.at[pl.ds(base, N_PER_W)], idx_v)

    gbuf, wbuf, gsem, wsem = (g0, g1), (w0, w1), (gs0, gs1), (ws0, ws1)

    def g_dma(g, b):
        return pltpu.make_async_copy(
            table_hbm.at[idx_v.at[pl.ds(g * K, K)]], gbuf[b], gsem[b])

    def w_dma(g, b):
        return pltpu.make_async_copy(
            wbuf[b], out_hbm.at[pl.ds(base + g * K, K)], wsem[b])

    g_dma(0, 0).start()
    g_dma(1, 1).start()

    def pair(g2, carry):
        for b in range(2):
            g = 2 * g2 + b
            g_dma(g, b).wait()

            @pl.when(g2 > 0)
            def _wait_prev_write():
                w_dma(g - 2, b).wait()

            src, dst = gbuf[b], wbuf[b]

            @plsc.parallel_loop(0, K, step=ROWS_PER_STEP, unroll=2)
            def _scale(i):
                for r in range(ROWS_PER_STEP):
                    for j in range(D // 16):
                        sl = pl.ds(j * 16, 16)
                        dst[i + r, sl] = src[i + r, sl] * SCALE

            w_dma(g, b).start()

            @pl.when(g + 2 < NCHUNK)
            def _next_gather():
                g_dma(g + 2, b).start()

        return carry

    lax.fori_loop(0, NPAIR, pair, 0)
    w_dma(NCHUNK - 2, 0).wait()
    w_dma(NCHUNK - 1, 1).wait()


@jax.jit
def kernel(tokens, table):
    idx = tokens.reshape(-1).astype(jnp.int32)
    call = pl.kernel(
        _sc_body,
        mesh=plsc.VectorSubcoreMesh(core_axis_name="c", subcore_axis_name="s"),
        out_type=jax.ShapeDtypeStruct((N, D), jnp.float32),
        scratch_types=[
            pltpu.VMEM((N_PER_W,), jnp.int32),
            pltpu.VMEM((K, D), jnp.float32),
            pltpu.VMEM((K, D), jnp.float32),
            pltpu.VMEM((K, D), jnp.float32),
            pltpu.VMEM((K, D), jnp.float32),
            pltpu.SemaphoreType.DMA,
            pltpu.SemaphoreType.DMA,
            pltpu.SemaphoreType.DMA,
            pltpu.SemaphoreType.DMA,
        ],
        compiler_params=pltpu.CompilerParams(use_tc_tiling_on_sc=False),
    )
    out = call(idx, table)
    return out.reshape(B, L, D)


# speedup vs baseline: 1.3090x; 1.1815x over previous
import math

import jax
import jax.numpy as jnp
from jax import lax
from jax.experimental import pallas as pl
from jax.experimental.pallas import tpu as pltpu
from jax.experimental.pallas import tpu_sc as plsc

B = 4096
L = 200
D = 64
V = 1000000
N = B * L
NC = 2
NS = 16
NW = NC * NS
N_PER_W = N // NW
K = 256
NCHUNK = N_PER_W // K
NPAIR = NCHUNK // 2
SCALE = math.sqrt(D)

DC = 400
DNCHUNK = V // DC


def _depad_body(table_hbm, s_hbm, buf2d, buf1d, sem):
    wid = lax.axis_index("s") * NC + lax.axis_index("c")

    def chunk(k, carry):
        cid = wid + k * NW

        @pl.when(cid < DNCHUNK)
        def _():
            r0 = cid * DC
            pltpu.async_copy(table_hbm.at[pl.ds(r0, DC)], buf2d, sem).wait()

            @plsc.parallel_loop(0, DC, step=4)
            def _repack(r):
                for rr in range(4):
                    for j in range(D // 16):
                        buf1d[pl.ds((r + rr) * D + j * 16, 16)] = (
                            buf2d[r + rr, pl.ds(j * 16, 16)]
                        )

            pltpu.async_copy(buf1d, s_hbm.at[pl.ds(r0 * D, DC * D)], sem).wait()

        return carry

    lax.fori_loop(0, (DNCHUNK + NW - 1) // NW, chunk, 0)


def _gather_body(tokens_hbm, table_hbm, out_hbm,
                 idx_v, g0, g1, w0, w1, gs0, gs1, ws0, ws1):
    wid = lax.axis_index("s") * NC + lax.axis_index("c")
    base = wid * N_PER_W
    pltpu.sync_copy(tokens_hbm.at[pl.ds(base, N_PER_W)], idx_v)

    gbuf, wbuf, gsem, wsem = (g0, g1), (w0, w1), (gs0, gs1), (ws0, ws1)

    def g_dma(g, b):
        return pltpu.make_async_copy(
            table_hbm.at[idx_v.at[pl.ds(g * K, K)]], gbuf[b], gsem[b])

    def w_dma(g, b):
        return pltpu.make_async_copy(
            wbuf[b], out_hbm.at[pl.ds(base + g * K, K), pl.ds(0, D)], wsem[b])

    g_dma(0, 0).start()
    g_dma(1, 1).start()

    def pair(g2, carry):
        for b in range(2):
            g = 2 * g2 + b
            g_dma(g, b).wait()

            @pl.when(g2 > 0)
            def _wait_prev_write():
                w_dma(g - 2, b).wait()

            src, dst = gbuf[b], wbuf[b]

            @plsc.parallel_loop(0, K, step=8, unroll=2)
            def _scale(i):
                for r in range(8):
                    for j in range(D // 16):
                        sl = pl.ds(j * 16, 16)
                        dst[i + r, sl] = src[i + r, sl] * SCALE

            w_dma(g, b).start()

            @pl.when(g + 2 < NCHUNK)
            def _next_gather():
                g_dma(g + 2, b).start()

        return carry

    lax.fori_loop(0, NPAIR, pair, 0)
    w_dma(NCHUNK - 2, 0).wait()
    w_dma(NCHUNK - 1, 1).wait()


@jax.jit
def kernel(tokens, table):
    idx = tokens.reshape(-1).astype(jnp.int32)

    depad = pl.kernel(
        _depad_body,
        mesh=plsc.VectorSubcoreMesh(core_axis_name="c", subcore_axis_name="s"),
        out_type=jax.ShapeDtypeStruct((V * D,), jnp.float32),
        scratch_types=[
            pltpu.VMEM((DC, D), jnp.float32),
            pltpu.VMEM((DC * D,), jnp.float32),
            pltpu.SemaphoreType.DMA,
        ],
        compiler_params=pltpu.CompilerParams(use_tc_tiling_on_sc=True),
    )
    staged = depad(table).reshape(V, D)

    gather = pl.kernel(
        _gather_body,
        mesh=plsc.VectorSubcoreMesh(core_axis_name="c", subcore_axis_name="s"),
        out_type=jax.ShapeDtypeStruct((N, 2 * D), jnp.float32),
        scratch_types=[
            pltpu.VMEM((N_PER_W,), jnp.int32),
            pltpu.VMEM((K, D), jnp.float32),
            pltpu.VMEM((K, D), jnp.float32),
            pltpu.VMEM((K, D), jnp.float32),
            pltpu.VMEM((K, D), jnp.float32),
            pltpu.SemaphoreType.DMA,
            pltpu.SemaphoreType.DMA,
            pltpu.SemaphoreType.DMA,
            pltpu.SemaphoreType.DMA,
        ],
        compiler_params=pltpu.CompilerParams(use_tc_tiling_on_sc=False),
    )
    out = gather(idx, staged)
    return out[:, :D].reshape(B, L, D)


# speedup vs baseline: 1.6889x; 1.2903x over previous
import math

import jax
import jax.numpy as jnp
from jax import lax
from jax.experimental import layout as jex_layout
from jax.experimental import pallas as pl
from jax.experimental.pallas import tpu as pltpu
from jax.experimental.pallas import tpu_sc as plsc

B = 4096
L = 200
D = 64
V = 1000000
N = B * L
NC = 2
NS = 16
NW = NC * NS
N_PER_W = N // NW
K = 256
NCHUNK = N_PER_W // K
NPAIR = NCHUNK // 2
SCALE = math.sqrt(D)

DC = 200
DNCHUNK = V // DC
DNK_MAX = (DNCHUNK + NW - 1) // NW


def _depad_body(table_hbm, s_hbm, r0_, r1_, p0_, p1_, rs0, rs1, ps0, ps1):
    wid = lax.axis_index("s") * NC + lax.axis_index("c")
    nk = jnp.where(wid < DNCHUNK - (DNK_MAX - 1) * NW, DNK_MAX, DNK_MAX - 1)
    rbuf, pbuf, rsem, psem = (r0_, r1_), (p0_, p1_), (rs0, rs1), (ps0, ps1)

    def cid_of(k):
        return wid + k * NW

    def r_dma(k, b):
        return pltpu.make_async_copy(
            table_hbm.at[pl.ds(cid_of(k) * (DC // 8), DC // 8)], rbuf[b],
            rsem[b])

    def w_dma(k, b):
        return pltpu.make_async_copy(
            pbuf[b], s_hbm.at[pl.ds(cid_of(k) * DC * D, DC * D)], psem[b])

    def step(k, b):
        @pl.when(k < nk)
        def _read():
            r_dma(k, b).start()

        c = k - 1
        bb = 1 - b

        @pl.when((c >= 0) & (c < nk))
        def _process():
            r_dma(c, bb).wait()

            @pl.when(c >= 2)
            def _wait_prev_write():
                w_dma(c - 2, bb).wait()

            src, dst = rbuf[bb], pbuf[bb]

            @plsc.parallel_loop(0, DC // 8, step=1)
            def _repack(r):
                for rr in range(8):
                    for j in range(D // 16):
                        dst[pl.ds((r * 8 + rr) * D + j * 16, 16)] = (
                            src[r, rr, pl.ds(j * 16, 16)]
                        )

            w_dma(c, bb).start()

    def pair(g, carry):
        step(2 * g, 0)
        step(2 * g + 1, 1)
        return carry

    lax.fori_loop(0, (DNK_MAX + 2) // 2 + 1, pair, 0)
    w_dma(nk - 2, 0).wait()
    w_dma(nk - 2, 1).wait()


def _gather_body(tokens_hbm, table_hbm, out_hbm,
                 idx_v, g0, g1, w0, w1, gs0, gs1, ws0, ws1):
    wid = lax.axis_index("s") * NC + lax.axis_index("c")
    base = wid * N_PER_W
    pltpu.sync_copy(tokens_hbm.at[pl.ds(base, N_PER_W)], idx_v)

    gbuf, wbuf, gsem, wsem = (g0, g1), (w0, w1), (gs0, gs1), (ws0, ws1)

    def g_dma(g, b):
        return pltpu.make_async_copy(
            table_hbm.at[idx_v.at[pl.ds(g * K, K)]], gbuf[b], gsem[b])

    def w_dma(g, b):
        return pltpu.make_async_copy(
            wbuf[b], out_hbm.at[pl.ds(base + g * K, K), pl.ds(0, D)], wsem[b])

    g_dma(0, 0).start()
    g_dma(1, 1).start()

    def pair(g2, carry):
        for b in range(2):
            g = 2 * g2 + b
            g_dma(g, b).wait()

            @pl.when(g2 > 0)
            def _wait_prev_write():
                w_dma(g - 2, b).wait()

            src, dst = gbuf[b], wbuf[b]

            @plsc.parallel_loop(0, K, step=8, unroll=2)
            def _scale(i):
                for r in range(8):
                    for j in range(D // 16):
                        sl = pl.ds(j * 16, 16)
                        dst[i + r, sl] = src[i + r, sl] * SCALE

            w_dma(g, b).start()

            @pl.when(g + 2 < NCHUNK)
            def _next_gather():
                g_dma(g + 2, b).start()

        return carry

    lax.fori_loop(0, NPAIR, pair, 0)
    w_dma(NCHUNK - 2, 0).wait()
    w_dma(NCHUNK - 1, 1).wait()


@jax.jit
def kernel(tokens, table):
    idx = tokens.reshape(-1).astype(jnp.int32)

    depad = pl.kernel(
        _depad_body,
        mesh=plsc.VectorSubcoreMesh(core_axis_name="c", subcore_axis_name="s"),
        out_type=jax.ShapeDtypeStruct((V * D,), jnp.float32),
        scratch_types=[
            pltpu.VMEM((DC // 8, 8, D), jnp.float32),
            pltpu.VMEM((DC // 8, 8, D), jnp.float32),
            pltpu.VMEM((DC * D,), jnp.float32),
            pltpu.VMEM((DC * D,), jnp.float32),
            pltpu.SemaphoreType.DMA,
            pltpu.SemaphoreType.DMA,
            pltpu.SemaphoreType.DMA,
            pltpu.SemaphoreType.DMA,
        ],
        compiler_params=pltpu.CompilerParams(use_tc_tiling_on_sc=True),
    )
    table_rm = jex_layout.with_layout_constraint(
        table,
        jex_layout.Layout(major_to_minor=(1, 0), tiling=((8, 128),)),
    )
    staged = depad(table_rm.reshape(V // 8, 8, D)).reshape(V, D)

    gather = pl.kernel(
        _gather_body,
        mesh=plsc.VectorSubcoreMesh(core_axis_name="c", subcore_axis_name="s"),
        out_type=jax.ShapeDtypeStruct((N, 2 * D), jnp.float32),
        scratch_types=[
            pltpu.VMEM((N_PER_W,), jnp.int32),
            pltpu.VMEM((K, D), jnp.float32),
            pltpu.VMEM((K, D), jnp.float32),
            pltpu.VMEM((K, D), jnp.float32),
            pltpu.VMEM((K, D), jnp.float32),
            pltpu.SemaphoreType.DMA,
            pltpu.SemaphoreType.DMA,
            pltpu.SemaphoreType.DMA,
            pltpu.SemaphoreType.DMA,
        ],
        compiler_params=pltpu.CompilerParams(use_tc_tiling_on_sc=False),
    )
    out = gather(idx, staged)
    return out[:, :D].reshape(B, L, D)
